# baseline (speedup 1.0000x reference)
import functools
import jax
import jax.numpy as jnp
from jax import lax
from jax.experimental import pallas as pl
from jax.experimental.pallas import tpu as pltpu
from jax.experimental.pallas import tpu_sc as plsc

B = 1024
T = 200
NPH = 1001
NSG = 1000
NLG = 1000
PH_DIM = 128
SG_DIM = 16
LG_DIM = 8
NMEL = 80
RW = 96

NW = 32
BPW = B // NW
LANES = 16
UNROLL = 8


def _tc_precompute(pt_ref, st_ref, lt_ref, sid_ref, lid_ref, w_ref, bias_ref,
                   p_ref, base_ref):
    W = w_ref[...]
    hp = lax.Precision.HIGHEST
    p_ref[...] = jnp.dot(pt_ref[...], W[1:1 + PH_DIM],
                         preferred_element_type=jnp.float32, precision=hp)
    SW = jnp.dot(st_ref[...], W[1 + PH_DIM:1 + PH_DIM + SG_DIM],
                 preferred_element_type=jnp.float32, precision=hp)
    LW = jnp.dot(lt_ref[...], W[1 + PH_DIM + SG_DIM:],
                 preferred_element_type=jnp.float32, precision=hp)
    iota_s = lax.broadcasted_iota(jnp.int32, (B, NSG), 1)
    oh_s = (sid_ref[...] == iota_s).astype(jnp.float32)
    oh_l = (lid_ref[...] == iota_s).astype(jnp.float32)
    base = (jnp.dot(oh_s, SW, preferred_element_type=jnp.float32, precision=hp)
            + jnp.dot(oh_l, LW, preferred_element_type=jnp.float32, precision=hp)
            + bias_ref[...])
    base_ref[...] = base


def _precompute(phoneme_table, singer_table, language_table, sid, lid, W, bias):
    return pl.pallas_call(
        _tc_precompute,
        out_shape=[
            jax.ShapeDtypeStruct((NPH, NMEL), jnp.float32),
            jax.ShapeDtypeStruct((B, NMEL), jnp.float32),
        ],
    )(phoneme_table, singer_table, language_table, sid, lid, W, bias)


def _sc_body(p_hbm, w_hbm, base_hbm, f0_hbm, idx_hbm, out_hbm,
             p_loc, w0_loc, base_loc, f0a, idxa, stage0, stage1, osem0, osem1):
    stages = (stage0, stage1)
    wid = lax.axis_index("s") * 2 + lax.axis_index("c")
    b0 = wid * BPW

    pltpu.sync_copy(p_hbm, p_loc)
    pltpu.sync_copy(w_hbm.at[0], w0_loc)
    pltpu.sync_copy(base_hbm.at[pl.ds(b0 * NMEL, BPW * NMEL)], base_loc)
    pltpu.sync_copy(f0_hbm.at[wid], f0a)
    pltpu.sync_copy(idx_hbm.at[wid], idxa)

    iotav = lax.iota(jnp.int32, LANES)
    w0v = [w0_loc[pl.ds(16 * k, 16)] for k in range(5)]

    def fill(bl, buf):
        basev = [base_loc[pl.ds(bl * NMEL + 16 * k, 16)] for k in range(5)]
        tok0 = bl * T

        @plsc.parallel_loop(0, T, unroll=UNROLL)
        def tok_body(t):
            ts = jnp.full((LANES,), tok0 + t, dtype=jnp.int32)
            r = plsc.load_gather(idxa, [ts])
            f = plsc.load_gather(f0a, [ts])
            rbase = r * NMEL
            v = []
            for k in range(5):
                g5 = plsc.load_gather(p_loc, [rbase + (iotav + 16 * k)])
                v.append(g5 + (f * w0v[k] + basev[k]))
            row = t * (RW // 2)
            st = stages[buf]
            st[pl.ds(row, 16)] = plsc.bitcast(plsc.pack(
                v[0], v[1], format=plsc.PackFormat.INTERLEAVED), jnp.float32)
            st[pl.ds(row + 16, 16)] = plsc.bitcast(plsc.pack(
                v[2], v[3], format=plsc.PackFormat.INTERLEAVED), jnp.float32)
            st[pl.ds(row + 32, 16)] = plsc.bitcast(plsc.pack(
                v[4], v[4], format=plsc.PackFormat.INTERLEAVED), jnp.float32)

    RWT = T * RW // 2

    def out_row(b):
        return out_hbm.at[pl.ds(b * RWT, RWT)]

    fill(0, 0)
    pltpu.async_copy(stage0, out_row(b0), osem0)
    fill(1, 1)
    pltpu.async_copy(stage1, out_row(b0 + 1), osem1)

    def pair_body(i, c):
        b = b0 + 2 * i
        pltpu.make_async_copy(stage0, out_row(b), osem0).wait()
        fill(2 * i, 0)
        pltpu.async_copy(stage0, out_row(b), osem0)
        pltpu.make_async_copy(stage1, out_row(b + 1), osem1).wait()
        fill(2 * i + 1, 1)
        pltpu.async_copy(stage1, out_row(b + 1), osem1)
        return c

    lax.fori_loop(1, BPW // 2, pair_body, 0)
    pltpu.make_async_copy(stage0, out_row(b0), osem0).wait()
    pltpu.make_async_copy(stage1, out_row(b0 + 1), osem1).wait()


@functools.lru_cache(maxsize=1)
def _sc_lookup():
    mesh = plsc.VectorSubcoreMesh(core_axis_name="c", subcore_axis_name="s")
    return pl.kernel(
        _sc_body,
        out_type=jax.ShapeDtypeStruct((B * T * RW // 2,), jnp.float32),
        mesh=mesh,
        compiler_params=pltpu.CompilerParams(needs_layout_passes=False),
        scratch_types=[
            pltpu.VMEM((NPH * NMEL,), jnp.float32),
            pltpu.VMEM((NMEL,), jnp.float32),
            pltpu.VMEM((BPW * NMEL,), jnp.float32),
            pltpu.VMEM((BPW * T,), jnp.float32),
            pltpu.VMEM((BPW * T,), jnp.int32),
            pltpu.VMEM((T * RW // 2,), jnp.float32),
            pltpu.VMEM((T * RW // 2,), jnp.float32),
            pltpu.SemaphoreType.DMA,
            pltpu.SemaphoreType.DMA,
        ],
    )


_PERM = (
    [2 * i for i in range(16)] + [2 * i + 1 for i in range(16)]
    + [32 + 2 * i for i in range(16)] + [33 + 2 * i for i in range(16)]
    + list(range(64, 80))
)
_TAIL = [64 + 2 * i for i in range(16)]


def kernel(f0, phoneme_seq, singer_id, language_id, phoneme_table,
           singer_table, language_table, W, b):
    idx = phoneme_seq.astype(jnp.int32)
    sid = singer_id.astype(jnp.int32).reshape(B, 1)
    lid = language_id.astype(jnp.int32).reshape(B, 1)
    perm = jnp.array(_PERM, dtype=jnp.int32)
    Wp = W[:, perm]
    bias = b[perm].reshape(1, NMEL)

    P, base = _precompute(phoneme_table, singer_table, language_table,
                          sid, lid, Wp, bias)

    out = _sc_lookup()(P.reshape(-1), Wp, base.reshape(-1),
                       f0.reshape(NW, BPW * T), idx.reshape(NW, BPW * T))
    out_bf = lax.bitcast_convert_type(out, jnp.bfloat16)
    out3 = out_bf.reshape(B, T, RW)
    res = jnp.concatenate([out3[:, :, :64], out3[:, :, _TAIL[0]:RW:2]], axis=-1)
    return res.astype(jnp.float32)


# speedup vs baseline: 2.7449x; 2.7449x over previous
import functools
import jax
import jax.numpy as jnp
from jax import lax
from jax.experimental import pallas as pl
from jax.experimental.pallas import tpu as pltpu
from jax.experimental.pallas import tpu_sc as plsc

B = 1024
T = 200
NPH = 1001
NSG = 1000
NLG = 1000
PH_DIM = 128
SG_DIM = 16
LG_DIM = 8
NMEL = 80

NW = 32
HALF = B // 2
BPW = HALF // NW
LANES = 16
UNROLL = 8


def _tc_precompute(pt_ref, st_ref, lt_ref, sid_ref, lid_ref, w_ref, bias_ref,
                   p_ref, base_ref):
    W = w_ref[...]
    hp = lax.Precision.HIGHEST
    p_ref[...] = jnp.dot(pt_ref[...], W[1:1 + PH_DIM],
                         preferred_element_type=jnp.float32, precision=hp)
    SW = jnp.dot(st_ref[...], W[1 + PH_DIM:1 + PH_DIM + SG_DIM],
                 preferred_element_type=jnp.float32, precision=hp)
    LW = jnp.dot(lt_ref[...], W[1 + PH_DIM + SG_DIM:],
                 preferred_element_type=jnp.float32, precision=hp)
    iota_s = lax.broadcasted_iota(jnp.int32, (B, NSG), 1)
    oh_s = (sid_ref[...] == iota_s).astype(jnp.float32)
    oh_l = (lid_ref[...] == iota_s).astype(jnp.float32)
    base = (jnp.dot(oh_s, SW, preferred_element_type=jnp.float32, precision=hp)
            + jnp.dot(oh_l, LW, preferred_element_type=jnp.float32, precision=hp)
            + bias_ref[...])
    base_ref[...] = base


def _precompute(phoneme_table, singer_table, language_table, sid, lid, W, bias):
    return pl.pallas_call(
        _tc_precompute,
        out_shape=[
            jax.ShapeDtypeStruct((NPH, NMEL), jnp.float32),
            jax.ShapeDtypeStruct((B, NMEL), jnp.float32),
        ],
    )(phoneme_table, singer_table, language_table, sid, lid, W, bias)


def _sc_body(p_hbm, w_hbm, base_hbm, f0_hbm, idx_hbm, out_hbm,
             p_loc, w0_loc, base_loc, f0a, idxa, out_stage, osem0, osem1):
    wid = lax.axis_index("s") * 2 + lax.axis_index("c")
    b0 = wid * BPW

    pltpu.sync_copy(p_hbm, p_loc)
    pltpu.sync_copy(w_hbm.at[0], w0_loc)
    pltpu.sync_copy(base_hbm.at[pl.ds(b0 * NMEL, BPW * NMEL)], base_loc)
    pltpu.sync_copy(f0_hbm.at[wid], f0a)
    pltpu.sync_copy(idx_hbm.at[wid], idxa)

    iotav = lax.iota(jnp.int32, LANES)
    w0v = [w0_loc[pl.ds(16 * k, 16)] for k in range(5)]

    def fill(bl, buf):
        basev = [base_loc[pl.ds(bl * NMEL + 16 * k, 16)] for k in range(5)]
        tok0 = bl * T

        @plsc.parallel_loop(0, T, unroll=UNROLL)
        def tok_body(t):
            ts = jnp.full((LANES,), tok0 + t, dtype=jnp.int32)
            r = plsc.load_gather(idxa, [ts])
            f = plsc.load_gather(f0a, [ts])
            rbase = r * NMEL
            for k in range(5):
                g5 = plsc.load_gather(p_loc, [rbase + (iotav + 16 * k)])
                out_stage[buf, pl.ds(t * NMEL + 16 * k, 16)] = (
                    g5 + (f * w0v[k] + basev[k]))

    fill(0, 0)
    pltpu.async_copy(out_stage.at[0], out_hbm.at[b0], osem0)
    fill(1, 1)
    pltpu.async_copy(out_stage.at[1], out_hbm.at[b0 + 1], osem1)

    def pair_body(i, c):
        b = b0 + 2 * i
        pltpu.make_async_copy(out_stage.at[0], out_hbm.at[b], osem0).wait()
        fill(2 * i, 0)
        pltpu.async_copy(out_stage.at[0], out_hbm.at[b], osem0)
        pltpu.make_async_copy(out_stage.at[1], out_hbm.at[b + 1], osem1).wait()
        fill(2 * i + 1, 1)
        pltpu.async_copy(out_stage.at[1], out_hbm.at[b + 1], osem1)
        return c

    lax.fori_loop(1, BPW // 2, pair_body, 0)
    pltpu.make_async_copy(out_stage.at[0], out_hbm.at[b0], osem0).wait()
    pltpu.make_async_copy(out_stage.at[1], out_hbm.at[b0 + 1], osem1).wait()


@functools.lru_cache(maxsize=1)
def _sc_lookup():
    mesh = plsc.VectorSubcoreMesh(core_axis_name="c", subcore_axis_name="s")
    return pl.kernel(
        _sc_body,
        out_type=jax.ShapeDtypeStruct((HALF, T * NMEL), jnp.float32),
        mesh=mesh,
        compiler_params=pltpu.CompilerParams(needs_layout_passes=False),
        scratch_types=[
            pltpu.VMEM((NPH * NMEL,), jnp.float32),
            pltpu.VMEM((NMEL,), jnp.float32),
            pltpu.VMEM((BPW * NMEL,), jnp.float32),
            pltpu.VMEM((BPW * T,), jnp.float32),
            pltpu.VMEM((BPW * T,), jnp.int32),
            pltpu.VMEM((2, T * NMEL), jnp.float32),
            pltpu.SemaphoreType.DMA,
            pltpu.SemaphoreType.DMA,
        ],
    )


def kernel(f0, phoneme_seq, singer_id, language_id, phoneme_table,
           singer_table, language_table, W, b):
    idx = phoneme_seq.astype(jnp.int32)
    sid = singer_id.astype(jnp.int32).reshape(B, 1)
    lid = language_id.astype(jnp.int32).reshape(B, 1)
    bias = b.reshape(1, NMEL)

    P, base = _precompute(phoneme_table, singer_table, language_table,
                          sid, lid, W, bias)

    pf = P.reshape(-1)
    sc = _sc_lookup()
    halves = []
    for h in range(2):
        sl = slice(h * HALF, (h + 1) * HALF)
        outh = sc(pf, W, base[sl].reshape(-1),
                  f0[sl].reshape(NW, BPW * T), idx[sl].reshape(NW, BPW * T))
        halves.append(outh.reshape(HALF, T, NMEL))
    return jnp.concatenate(halves, axis=0)


# speedup vs baseline: 3.3688x; 1.2273x over previous
import functools
import jax
import jax.numpy as jnp
from jax import lax
from jax.experimental import pallas as pl
from jax.experimental.pallas import tpu as pltpu
from jax.experimental.pallas import tpu_sc as plsc

B = 1024
T = 200
NPH = 1001
NSG = 1000
NLG = 1000
PH_DIM = 128
SG_DIM = 16
LG_DIM = 8
NMEL = 80

NW = 32
BPW = B // NW
LANES = 16
UNROLL = 8


def _tc_precompute(pt_ref, st_ref, lt_ref, sid_ref, lid_ref, w_ref, bias_ref,
                   p_ref, base_ref):
    W = w_ref[...]
    hp = lax.Precision.HIGHEST
    p_ref[...] = jnp.dot(pt_ref[...], W[1:1 + PH_DIM],
                         preferred_element_type=jnp.float32, precision=hp)
    SW = jnp.dot(st_ref[...], W[1 + PH_DIM:1 + PH_DIM + SG_DIM],
                 preferred_element_type=jnp.float32, precision=hp)
    LW = jnp.dot(lt_ref[...], W[1 + PH_DIM + SG_DIM:],
                 preferred_element_type=jnp.float32, precision=hp)
    iota_s = lax.broadcasted_iota(jnp.int32, (B, NSG), 1)
    oh_s = (sid_ref[...] == iota_s).astype(jnp.float32)
    oh_l = (lid_ref[...] == iota_s).astype(jnp.float32)
    base = (jnp.dot(oh_s, SW, preferred_element_type=jnp.float32, precision=hp)
            + jnp.dot(oh_l, LW, preferred_element_type=jnp.float32, precision=hp)
            + bias_ref[...])
    base_ref[...] = base


def _precompute(phoneme_table, singer_table, language_table, sid, lid, W, bias):
    return pl.pallas_call(
        _tc_precompute,
        out_shape=[
            jax.ShapeDtypeStruct((NPH, NMEL), jnp.float32),
            jax.ShapeDtypeStruct((B, NMEL), jnp.float32),
        ],
    )(phoneme_table, singer_table, language_table, sid, lid, W, bias)


def _sc_body(p_hbm, w_hbm, base_hbm, f0_hbm, idx_hbm, out_hbm,
             p_loc, w0_loc, base_loc, f0a, idxa, out_stage, osem0, osem1):
    wid = lax.axis_index("s") * 2 + lax.axis_index("c")
    b0 = wid * BPW

    pltpu.sync_copy(p_hbm, p_loc)
    pltpu.sync_copy(w_hbm.at[0], w0_loc)
    pltpu.sync_copy(base_hbm.at[pl.ds(b0 * NMEL, BPW * NMEL)], base_loc)
    pltpu.sync_copy(f0_hbm.at[wid], f0a)
    pltpu.sync_copy(idx_hbm.at[wid], idxa)

    iotav = lax.iota(jnp.int32, LANES)
    w0v = [w0_loc[pl.ds(16 * k, 16)] for k in range(5)]

    def fill(bl, buf):
        basev = [base_loc[pl.ds(bl * NMEL + 16 * k, 16)] for k in range(5)]
        tok0 = bl * T

        @plsc.parallel_loop(0, T, unroll=UNROLL)
        def tok_body(t):
            ts = jnp.full((LANES,), tok0 + t, dtype=jnp.int32)
            r = plsc.load_gather(idxa, [ts])
            f = plsc.load_gather(f0a, [ts])
            rbase = r * NMEL
            for k in range(5):
                g5 = plsc.load_gather(p_loc, [rbase + (iotav + 16 * k)])
                out_stage[buf, pl.ds(t * NMEL + 16 * k, 16)] = (
                    g5 + (f * w0v[k] + basev[k]))

    fill(0, 0)
    pltpu.async_copy(out_stage.at[0], out_hbm.at[b0], osem0)
    fill(1, 1)
    pltpu.async_copy(out_stage.at[1], out_hbm.at[b0 + 1], osem1)

    def pair_body(i, c):
        b = b0 + 2 * i
        pltpu.make_async_copy(out_stage.at[0], out_hbm.at[b], osem0).wait()
        fill(2 * i, 0)
        pltpu.async_copy(out_stage.at[0], out_hbm.at[b], osem0)
        pltpu.make_async_copy(out_stage.at[1], out_hbm.at[b + 1], osem1).wait()
        fill(2 * i + 1, 1)
        pltpu.async_copy(out_stage.at[1], out_hbm.at[b + 1], osem1)
        return c

    lax.fori_loop(1, BPW // 2, pair_body, 0)
    pltpu.make_async_copy(out_stage.at[0], out_hbm.at[b0], osem0).wait()
    pltpu.make_async_copy(out_stage.at[1], out_hbm.at[b0 + 1], osem1).wait()


@functools.lru_cache(maxsize=1)
def _sc_lookup():
    mesh = plsc.VectorSubcoreMesh(core_axis_name="c", subcore_axis_name="s")
    return pl.kernel(
        _sc_body,
        out_type=jax.ShapeDtypeStruct((B, T * NMEL), jnp.float32),
        mesh=mesh,
        compiler_params=pltpu.CompilerParams(needs_layout_passes=False),
        scratch_types=[
            pltpu.VMEM((NPH * NMEL,), jnp.float32),
            pltpu.VMEM((NMEL,), jnp.float32),
            pltpu.VMEM((BPW * NMEL,), jnp.float32),
            pltpu.VMEM((BPW * T,), jnp.float32),
            pltpu.VMEM((BPW * T,), jnp.int32),
            pltpu.VMEM((2, T * NMEL), jnp.float32),
            pltpu.SemaphoreType.DMA,
            pltpu.SemaphoreType.DMA,
        ],
    )


def kernel(f0, phoneme_seq, singer_id, language_id, phoneme_table,
           singer_table, language_table, W, b):
    idx = phoneme_seq.astype(jnp.int32)
    sid = singer_id.astype(jnp.int32).reshape(B, 1)
    lid = language_id.astype(jnp.int32).reshape(B, 1)
    bias = b.reshape(1, NMEL)

    P, base = _precompute(phoneme_table, singer_table, language_table,
                          sid, lid, W, bias)

    out = _sc_lookup()(P.reshape(-1), W, base.reshape(-1),
                       f0.reshape(NW, BPW * T), idx.reshape(NW, BPW * T))
    return out.reshape(B, T, NMEL)
